# baseline (speedup 1.0000x reference)
import functools

import jax
import jax.numpy as jnp
from jax import lax
from jax.experimental import pallas as pl
from jax.experimental.pallas import tpu as pltpu
from jax.experimental.pallas import tpu_sc as plsc

_BB = 1024
_K = 8


def _tc_body(x_ref, w_ref, b_ref, n_ref, wout_ref, lout_ref, *, E):
    acc = jnp.dot(x_ref[...], w_ref[...], preferred_element_type=jnp.float32)
    acc = acc + b_ref[...]
    clean = acc[:, :E]
    raw = acc[:, E:]
    std = jnp.maximum(raw, 0.0) + jnp.log1p(jnp.exp(-jnp.abs(raw)))
    logits = clean + n_ref[...] * std
    lout_ref[...] = logits
    m = jnp.max(logits, axis=-1, keepdims=True)
    e = jnp.exp(logits - m)
    s = jnp.sum(e, axis=-1, keepdims=True)
    wout_ref[...] = e * (1.0 / s)


def _tc_gating(x, W, b2, noise, E):
    B, D = x.shape
    return pl.pallas_call(
        functools.partial(_tc_body, E=E),
        grid=(B // _BB,),
        in_specs=[
            pl.BlockSpec((_BB, D), lambda i: (i, 0)),
            pl.BlockSpec((D, 2 * E), lambda i: (0, 0)),
            pl.BlockSpec((1, 2 * E), lambda i: (0, 0)),
            pl.BlockSpec((_BB, E), lambda i: (i, 0)),
        ],
        out_specs=[
            pl.BlockSpec((_BB, E), lambda i: (i, 0)),
            pl.BlockSpec((_BB, E), lambda i: (i, 0)),
        ],
        out_shape=[
            jax.ShapeDtypeStruct((B, E), jnp.float32),
            jax.ShapeDtypeStruct((B, E), jnp.float32),
        ],
        compiler_params=pltpu.CompilerParams(
            dimension_semantics=("parallel",)),
    )(x, W, b2, noise)


def _merge_top16(ka, va, kb, vb):
    kbr = jnp.flip(kb)
    vbr = jnp.flip(vb)
    take_a = ka >= kbr
    km = jnp.where(take_a, ka, kbr)
    vm = jnp.where(take_a, va, vbr)
    return plsc.sort_key_val(km, vm, descending=True)


def _make_sc_topk(B, E):
    info = plsc.get_sparse_core_info()
    NC, NS = info.num_cores, info.num_subcores
    NW = NC * NS
    rows = B // NW
    mesh = plsc.VectorSubcoreMesh(core_axis_name="c", subcore_axis_name="s")

    @functools.partial(
        pl.kernel, mesh=mesh,
        out_type=[jax.ShapeDtypeStruct((B * _K,), jnp.float32),
                  jax.ShapeDtypeStruct((B * _K,), jnp.int32)],
        scratch_types=[
            pltpu.VMEM((rows, E), jnp.float32),
            pltpu.VMEM((rows * _K + 8,), jnp.float32),
            pltpu.VMEM((rows * _K + 8,), jnp.int32),
        ],
        compiler_params=pltpu.CompilerParams(needs_layout_passes=False),
    )
    def sc_topk(w_hbm, vout_hbm, iout_hbm, wv, vv, iv):
        wid = lax.axis_index("s") * NC + lax.axis_index("c")
        base = wid * rows
        pltpu.sync_copy(w_hbm.at[pl.ds(base, rows)], wv)

        lane = lax.iota(jnp.int32, 16)
        first8 = lane < 8

        @plsc.parallel_loop(0, rows, unroll=4)
        def row_body(r):
            ks, vs = [], []
            for g in range(E // 16):
                key = wv[r, pl.ds(g * 16, 16)]
                idx = lane + (g * 16)
                k_s, v_s = plsc.sort_key_val(key, idx, descending=True)
                ks.append(k_s)
                vs.append(v_s)
            k01, v01 = _merge_top16(ks[0], vs[0], ks[1], vs[1])
            k23, v23 = _merge_top16(ks[2], vs[2], ks[3], vs[3])
            kf, vf = _merge_top16(k01, v01, k23, v23)
            plsc.store_compressed(vv.at[pl.ds(r * _K, 16)], kf, mask=first8)
            plsc.store_compressed(iv.at[pl.ds(r * _K, 16)], vf, mask=first8)

        pltpu.sync_copy(vv.at[pl.ds(0, rows * _K)],
                        vout_hbm.at[pl.ds(base * _K, rows * _K)])
        pltpu.sync_copy(iv.at[pl.ds(0, rows * _K)],
                        iout_hbm.at[pl.ds(base * _K, rows * _K)])

    return sc_topk


def kernel(x, Wg, bg, Wn, bn, noise, top_k):
    B, D = x.shape
    E = Wg.shape[0]
    W = jnp.concatenate([Wg, Wn], axis=0).T
    b2 = jnp.concatenate([bg, bn])[None, :]
    H = B // 2
    sc = _make_sc_topk(H, E)
    w0, l0 = _tc_gating(x[:H], W, b2, noise[:H], E)
    tv0, ti0 = sc(w0)
    w1, l1 = _tc_gating(x[H:], W, b2, noise[H:], E)
    tv1, ti1 = sc(w1)
    weights = jnp.concatenate([w0, w1], axis=0)
    logits = jnp.concatenate([l0, l1], axis=0)
    tv = jnp.concatenate([tv0.reshape(H, _K), tv1.reshape(H, _K)], axis=0)
    ti = jnp.concatenate([ti0.reshape(H, _K), ti1.reshape(H, _K)], axis=0)
    return (weights, logits, tv, ti)


# speedup vs baseline: 2.4266x; 2.4266x over previous
import functools

import jax
import jax.numpy as jnp
from jax.experimental import pallas as pl
from jax.experimental.pallas import tpu as pltpu

_BB = 1024
_K = 8
_NEG = -3.0e38


def _epilogue(acc_ref, b_ref, n_ref, wout_ref, lout_ref, vout_ref, iout_ref,
              E):
    acc = acc_ref[...] + b_ref[...]
    clean = acc[:, :E]
    raw = acc[:, E:]
    std = jnp.maximum(raw, 0.0) + jnp.log1p(jnp.exp(-jnp.abs(raw)))
    logits = clean + n_ref[...] * std
    lout_ref[...] = logits
    cols = jax.lax.broadcasted_iota(jnp.int32, logits.shape, 1)
    work = logits
    mxs, idxs = [], []
    for _ in range(_K):
        mx = jnp.max(work, axis=-1, keepdims=True)
        am = jnp.argmax(work, axis=-1).astype(jnp.int32)[:, None]
        mxs.append(mx)
        idxs.append(am)
        work = jnp.where(cols == am, _NEG, work)
    m = mxs[0]
    e = jnp.exp(logits - m)
    s = jnp.sum(e, axis=-1, keepdims=True)
    inv_s = 1.0 / s
    wout_ref[...] = e * inv_s
    tl = jnp.concatenate(mxs, axis=1)
    vout_ref[...] = jnp.exp(tl - m) * inv_s
    iout_ref[...] = jnp.concatenate(idxs, axis=1)


def _body(x_ref, w_ref, b_ref, n_ref, wout_ref, lout_ref, vout_ref, iout_ref,
          acc0_ref, acc1_ref, *, E, nb):
    i = pl.program_id(0)

    @pl.when(i < nb)
    def _matmul():
        mm = jnp.dot(x_ref[...], w_ref[...],
                     preferred_element_type=jnp.float32)

        @pl.when(i % 2 == 0)
        def _w0():
            acc0_ref[...] = mm

        @pl.when(i % 2 == 1)
        def _w1():
            acc1_ref[...] = mm

    @pl.when(i > 0)
    def _epi():
        @pl.when(i % 2 == 1)
        def _e0():
            _epilogue(acc0_ref, b_ref, n_ref, wout_ref, lout_ref, vout_ref,
                      iout_ref, E)

        @pl.when(i % 2 == 0)
        def _e1():
            _epilogue(acc1_ref, b_ref, n_ref, wout_ref, lout_ref, vout_ref,
                      iout_ref, E)


def kernel(x, Wg, bg, Wn, bn, noise, top_k):
    B, D = x.shape
    E = Wg.shape[0]
    W = jnp.concatenate([Wg, Wn], axis=0).T
    b2 = jnp.concatenate([bg, bn])[None, :]
    nb = B // _BB
    grid = (nb + 1,)

    def x_map(i):
        return (jnp.minimum(i, nb - 1), 0)

    def prev_map(i):
        return (jnp.maximum(i - 1, 0), 0)

    out = pl.pallas_call(
        functools.partial(_body, E=E, nb=nb),
        grid=grid,
        in_specs=[
            pl.BlockSpec((_BB, D), x_map),
            pl.BlockSpec((D, 2 * E), lambda i: (0, 0)),
            pl.BlockSpec((1, 2 * E), lambda i: (0, 0)),
            pl.BlockSpec((_BB, E), prev_map),
        ],
        out_specs=[
            pl.BlockSpec((_BB, E), prev_map),
            pl.BlockSpec((_BB, E), prev_map),
            pl.BlockSpec((_BB, _K), prev_map),
            pl.BlockSpec((_BB, _K), prev_map),
        ],
        out_shape=[
            jax.ShapeDtypeStruct((B, E), jnp.float32),
            jax.ShapeDtypeStruct((B, E), jnp.float32),
            jax.ShapeDtypeStruct((B, _K), jnp.float32),
            jax.ShapeDtypeStruct((B, _K), jnp.int32),
        ],
        scratch_shapes=[pltpu.VMEM((_BB, 2 * E), jnp.float32),
                        pltpu.VMEM((_BB, 2 * E), jnp.float32)],
        compiler_params=pltpu.CompilerParams(
            dimension_semantics=("arbitrary",)),
    )(x, W, b2, noise)
    weights, logits, top_k_vals, top_k_indices = out
    return (weights, logits, top_k_vals, top_k_indices)


# speedup vs baseline: 2.6089x; 1.0751x over previous
import functools

import jax
import jax.numpy as jnp
from jax.experimental import pallas as pl
from jax.experimental.pallas import tpu as pltpu

_BB = 1024
_K = 8
_NEG = -3.0e38


def _epilogue(acc_ref, b_ref, n_ref, wout_ref, lout_ref, vout_ref, iout_ref,
              E):
    acc = (acc_ref[...] + b_ref[...]).T
    clean = acc[:E, :]
    raw = acc[E:, :]
    std = jnp.maximum(raw, 0.0) + jnp.log1p(jnp.exp(-jnp.abs(raw)))
    logits = clean + n_ref[...].T * std
    lout_ref[...] = logits.T
    rows = jax.lax.broadcasted_iota(jnp.int32, logits.shape, 0)
    work = logits
    mxs, idxs = [], []
    for _ in range(_K):
        mx = jnp.max(work, axis=0, keepdims=True)
        am = jnp.argmax(work, axis=0).astype(jnp.int32)[None, :]
        mxs.append(mx)
        idxs.append(am)
        work = jnp.where(rows == am, _NEG, work)
    m = mxs[0]
    e = jnp.exp(logits - m)
    s = jnp.sum(e, axis=0, keepdims=True)
    inv_s = 1.0 / s
    wout_ref[...] = (e * inv_s).T
    tl = jnp.concatenate(mxs, axis=0)
    vout_ref[...] = (jnp.exp(tl - m) * inv_s).T
    iout_ref[...] = jnp.concatenate(idxs, axis=0).T


def _body(x_ref, w_ref, b_ref, n_ref, wout_ref, lout_ref, vout_ref, iout_ref,
          acc0_ref, acc1_ref, *, E, nb):
    i = pl.program_id(0)

    @pl.when(i < nb)
    def _matmul():
        mm = jnp.dot(x_ref[...], w_ref[...],
                     preferred_element_type=jnp.float32)

        @pl.when(i % 2 == 0)
        def _w0():
            acc0_ref[...] = mm

        @pl.when(i % 2 == 1)
        def _w1():
            acc1_ref[...] = mm

    @pl.when(i > 0)
    def _epi():
        @pl.when(i % 2 == 1)
        def _e0():
            _epilogue(acc0_ref, b_ref, n_ref, wout_ref, lout_ref, vout_ref,
                      iout_ref, E)

        @pl.when(i % 2 == 0)
        def _e1():
            _epilogue(acc1_ref, b_ref, n_ref, wout_ref, lout_ref, vout_ref,
                      iout_ref, E)


def kernel(x, Wg, bg, Wn, bn, noise, top_k):
    B, D = x.shape
    E = Wg.shape[0]
    W = jnp.concatenate([Wg, Wn], axis=0).T
    b2 = jnp.concatenate([bg, bn])[None, :]
    nb = B // _BB
    grid = (nb + 1,)

    def x_map(i):
        return (jnp.minimum(i, nb - 1), 0)

    def prev_map(i):
        return (jnp.maximum(i - 1, 0), 0)

    out = pl.pallas_call(
        functools.partial(_body, E=E, nb=nb),
        grid=grid,
        in_specs=[
            pl.BlockSpec((_BB, D), x_map),
            pl.BlockSpec((D, 2 * E), lambda i: (0, 0)),
            pl.BlockSpec((1, 2 * E), lambda i: (0, 0)),
            pl.BlockSpec((_BB, E), prev_map),
        ],
        out_specs=[
            pl.BlockSpec((_BB, E), prev_map),
            pl.BlockSpec((_BB, E), prev_map),
            pl.BlockSpec((_BB, _K), prev_map),
            pl.BlockSpec((_BB, _K), prev_map),
        ],
        out_shape=[
            jax.ShapeDtypeStruct((B, E), jnp.float32),
            jax.ShapeDtypeStruct((B, E), jnp.float32),
            jax.ShapeDtypeStruct((B, _K), jnp.float32),
            jax.ShapeDtypeStruct((B, _K), jnp.int32),
        ],
        scratch_shapes=[pltpu.VMEM((_BB, 2 * E), jnp.float32),
                        pltpu.VMEM((_BB, 2 * E), jnp.float32)],
        compiler_params=pltpu.CompilerParams(
            dimension_semantics=("arbitrary",)),
    )(x, W, b2, noise)
    weights, logits, top_k_vals, top_k_indices = out
    return (weights, logits, top_k_vals, top_k_indices)
